# Initial kernel scaffold; baseline (speedup 1.0000x reference)
import jax
import jax.numpy as jnp
from jax.experimental import pallas as pl


def kernel(preds, thresh, max_proposals):
    raise NotImplementedError("write your pallas kernel here")


# speedup vs baseline: 21.3439x; 21.3439x over previous
import functools

import jax
import jax.numpy as jnp
from jax import lax
from jax.experimental import pallas as pl


def _suppress_kernel(thresh_ref, x1_ref, y1_ref, x2_ref, y2_ref, area_ref,
                     sup_ref, *, n_real: int, rows: int):
    lane2 = lax.broadcasted_iota(jnp.int32, (rows, 128), 1)
    row2 = lax.broadcasted_iota(jnp.int32, (rows, 128), 0)
    pos = row2 * 128 + lane2
    sup_ref[:, :] = jnp.where(pos >= n_real, 1.0, 0.0)
    thresh = thresh_ref[0, 0]
    lane1 = lax.broadcasted_iota(jnp.int32, (1, 128), 1)

    def lane_body(l, r):
        srow = sup_ref[pl.ds(r, 1), :]
        onehot = lane1 == l
        s_i = jnp.sum(jnp.where(onehot, srow, 0.0))

        @pl.when(s_i == 0.0)
        def _():
            x1_i = jnp.sum(jnp.where(onehot, x1_ref[pl.ds(r, 1), :], 0.0))
            y1_i = jnp.sum(jnp.where(onehot, y1_ref[pl.ds(r, 1), :], 0.0))
            x2_i = jnp.sum(jnp.where(onehot, x2_ref[pl.ds(r, 1), :], 0.0))
            y2_i = jnp.sum(jnp.where(onehot, y2_ref[pl.ds(r, 1), :], 0.0))
            area_i = (x2_i - x1_i) * (y2_i - y1_i)
            xx1 = jnp.maximum(x1_i, x1_ref[:, :])
            yy1 = jnp.maximum(y1_i, y1_ref[:, :])
            xx2 = jnp.minimum(x2_i, x2_ref[:, :])
            yy2 = jnp.minimum(y2_i, y2_ref[:, :])
            w = jnp.maximum(xx2 - xx1, 0.0)
            h = jnp.maximum(yy2 - yy1, 0.0)
            inter = w * h
            iou = inter / (area_i + area_ref[:, :] - inter)
            i = r * 128 + l
            hit = (iou > thresh) & (pos > i)
            sup_ref[:, :] = jnp.where(hit, 1.0, sup_ref[:, :])

        return r

    def row_body(r, carry):
        srow = sup_ref[pl.ds(r, 1), :]
        any_kept = jnp.min(srow) < 0.5

        @pl.when(any_kept)
        def _():
            lax.fori_loop(0, 128, lane_body, r)

        return carry

    lax.fori_loop(0, rows, row_body, 0)


def kernel(preds, thresh, max_proposals):
    n = preds.shape[0]
    npad = ((n + 1023) // 1024) * 1024
    rows = npad // 128

    scores = preds[:, 4]
    order = jnp.argsort(-scores)
    b = preds[order]
    coords = jnp.zeros((npad, 4), jnp.float32).at[:n].set(b[:, :4])
    x1 = coords[:, 0].reshape(rows, 128)
    y1 = coords[:, 1].reshape(rows, 128)
    x2 = coords[:, 2].reshape(rows, 128)
    y2 = coords[:, 3].reshape(rows, 128)
    areas = (x2 - x1) * (y2 - y1)
    thresh_arr = jnp.asarray(thresh, jnp.float32).reshape(1, 1)

    sup = pl.pallas_call(
        functools.partial(_suppress_kernel, n_real=n, rows=rows),
        out_shape=jax.ShapeDtypeStruct((rows, 128), jnp.float32),
    )(thresh_arr, x1, y1, x2, y2, areas)

    supf = sup.reshape(-1)[:n]
    idx = jnp.arange(n, dtype=jnp.int32)
    keys = idx + supf.astype(jnp.int32) * n
    _, sel_pos = lax.top_k(-keys, 1000)
    keep1000 = order[sel_pos].astype(jnp.int32)
    sel = jnp.minimum(jnp.arange(1000), max_proposals - 1)
    keep = keep1000[sel]
    return preds[keep], keep
